# Initial kernel scaffold; baseline (speedup 1.0000x reference)
import jax
import jax.numpy as jnp
from jax.experimental import pallas as pl


def kernel(char_ids, embedding, conv_w, conv_b, proj_w, proj_b, gate_w, gate_b):
    raise NotImplementedError("write your pallas kernel here")


# speedup vs baseline: 1.9509x; 1.9509x over previous
import functools

import jax
import jax.numpy as jnp
from jax.experimental import pallas as pl
from jax.experimental.pallas import tpu as pltpu

_CHAR_EMBED = 50
_MAX_WORD_LEN = 21
_KSIZE = 5
_PAD_IDX = 0


def _fused_kernel(ids_ref, embt_ref, w2_ref, bc_ref, wpt_ref, bp_ref,
                  wgt_ref, bg_ref, eye_ref, o_ref, *,
                  ksize, t_out, vocab, tile_n, c_pad):
    l_tot = t_out + ksize - 1
    p = l_tot * tile_n
    m = t_out * tile_n

    ids = ids_ref[0]
    iota_v = jax.lax.broadcasted_iota(jnp.int32, (vocab, p), 0)
    onehot = (iota_v == ids).astype(jnp.float32)

    emb = jnp.dot(embt_ref[...], onehot,
                  preferred_element_type=jnp.float32)
    embb = emb.astype(jnp.bfloat16)

    x = jnp.concatenate(
        [embb[:, k * tile_n:k * tile_n + m] for k in range(ksize)],
        axis=0)

    conv = jnp.dot(w2_ref[...], x,
                   preferred_element_type=jnp.float32)

    pooled = conv[:, 0:tile_n]
    for t in range(1, t_out):
        pooled = jnp.maximum(pooled, conv[:, t * tile_n:(t + 1) * tile_n])

    cb = jnp.maximum(pooled + bc_ref[...], 0.0)
    cbb = cb.astype(jnp.bfloat16)

    dn = (((0,), (0,)), ((), ()))
    projt = jnp.maximum(
        jax.lax.dot_general(cbb, wpt_ref[...], dn,
                            preferred_element_type=jnp.float32) + bp_ref[...],
        0.0)
    gatet = jax.nn.sigmoid(
        jax.lax.dot_general(cbb, wgt_ref[...], dn,
                            preferred_element_type=jnp.float32) + bg_ref[...])
    convt = jax.lax.dot_general(cbb, eye_ref[...], dn,
                                preferred_element_type=jnp.float32)

    o_ref[...] = convt + gatet * (projt - convt)


def kernel(char_ids, embedding, conv_w, conv_b, proj_w, proj_b, gate_w,
           gate_b, *, tile_n=256):
    s_len, b_size, l = char_ids.shape
    assert l == _MAX_WORD_LEN
    n = s_len * b_size
    k = _KSIZE
    t_out = l - k + 1
    e = conv_w.shape[0]
    vcb = embedding.shape[0]
    c = embedding.shape[1]
    c_pad = 64

    n_pad = ((n + tile_n - 1) // tile_n) * tile_n
    nb = n_pad // tile_n
    ids = char_ids.reshape(n, l).astype(jnp.int32)
    if n_pad != n:
        ids = jnp.concatenate(
            [ids, jnp.full((n_pad - n, l), _PAD_IDX, dtype=jnp.int32)], axis=0)
    ids_l = ids.reshape(nb, tile_n, l).transpose(0, 2, 1).reshape(nb, 1, l * tile_n)

    embt = jnp.zeros((c_pad, vcb), jnp.float32).at[:c].set(embedding.T)
    w2 = jnp.zeros((e, k, c_pad), jnp.float32)
    w2 = w2.at[:, :, :c].set(jnp.transpose(conv_w, (0, 2, 1)))
    w2 = w2.reshape(e, k * c_pad).astype(jnp.bfloat16)
    bc = conv_b.reshape(e, 1)
    wpt = proj_w.T.astype(jnp.bfloat16)
    wgt = gate_w.T.astype(jnp.bfloat16)
    bp = proj_b.reshape(1, e)
    bg = gate_b.reshape(1, e)
    eye = jnp.eye(e, dtype=jnp.bfloat16)

    kern = functools.partial(_fused_kernel, ksize=k, t_out=t_out, vocab=vcb,
                             tile_n=tile_n, c_pad=c_pad)

    out = pl.pallas_call(
        kern,
        out_shape=jax.ShapeDtypeStruct((n_pad, e), jnp.float32),
        grid_spec=pltpu.PrefetchScalarGridSpec(
            num_scalar_prefetch=0,
            grid=(nb,),
            in_specs=[
                pl.BlockSpec((1, 1, l * tile_n), lambda i: (i, 0, 0)),
                pl.BlockSpec((c_pad, vcb), lambda i: (0, 0)),
                pl.BlockSpec((e, k * c_pad), lambda i: (0, 0)),
                pl.BlockSpec((e, 1), lambda i: (0, 0)),
                pl.BlockSpec((e, e), lambda i: (0, 0)),
                pl.BlockSpec((1, e), lambda i: (0, 0)),
                pl.BlockSpec((e, e), lambda i: (0, 0)),
                pl.BlockSpec((1, e), lambda i: (0, 0)),
                pl.BlockSpec((e, e), lambda i: (0, 0)),
            ],
            out_specs=pl.BlockSpec((tile_n, e), lambda i: (i, 0)),
        ),
        compiler_params=pltpu.CompilerParams(
            dimension_semantics=("parallel",),
            vmem_limit_bytes=64 * 1024 * 1024,
        ),
    )(ids_l, embt, w2, bc, wpt, bp, wgt, bg, eye)

    return out[:n].reshape(s_len, b_size, e)


# speedup vs baseline: 3.7938x; 1.9446x over previous
import functools

import jax
import jax.numpy as jnp
from jax.experimental import pallas as pl
from jax.experimental.pallas import tpu as pltpu

_CHAR_EMBED = 50
_MAX_WORD_LEN = 21
_KSIZE = 5
_PAD_IDX = 0


def _fused_kernel(ids_ref, embt_ref, w2t_ref, bc_ref, wpt_ref, bp_ref,
                  wgt_ref, bg_ref, o_ref, *,
                  ksize, t_out, vocab, tile_n, c_dim):
    p = (t_out + ksize - 1) * tile_n

    ids = ids_ref[0]
    iota_v = jax.lax.broadcasted_iota(jnp.int32, (vocab, p), 0)
    onehot = (iota_v == ids).astype(jnp.float32)

    emb = jnp.dot(embt_ref[...], onehot,
                  preferred_element_type=jnp.float32)
    embb = emb.astype(jnp.bfloat16)

    dn = (((0,), (0,)), ((), ()))
    pooled = None
    for t in range(t_out):
        xt = jnp.concatenate(
            [embb[:, (t + kk) * tile_n:(t + kk + 1) * tile_n]
             for kk in range(ksize)], axis=0)
        ct = jax.lax.dot_general(xt, w2t_ref[...], dn,
                                 preferred_element_type=jnp.float32)
        pooled = ct if pooled is None else jnp.maximum(pooled, ct)

    cb = jnp.maximum(pooled + bc_ref[...], 0.0)
    cbb = cb.astype(jnp.bfloat16)

    projt = jnp.maximum(
        jnp.dot(cbb, wpt_ref[...],
                preferred_element_type=jnp.float32) + bp_ref[...], 0.0)
    gatet = jax.nn.sigmoid(
        jnp.dot(cbb, wgt_ref[...],
                preferred_element_type=jnp.float32) + bg_ref[...])

    o_ref[...] = cb + gatet * (projt - cb)


def kernel(char_ids, embedding, conv_w, conv_b, proj_w, proj_b, gate_w,
           gate_b, *, tile_n=1024):
    s_len, b_size, l = char_ids.shape
    assert l == _MAX_WORD_LEN
    n = s_len * b_size
    k = _KSIZE
    t_out = l - k + 1
    e = conv_w.shape[0]
    vcb = embedding.shape[0]
    c = embedding.shape[1]

    n_pad = ((n + tile_n - 1) // tile_n) * tile_n
    nb = n_pad // tile_n
    ids = char_ids.reshape(n, l).astype(jnp.int32)
    if n_pad != n:
        ids = jnp.concatenate(
            [ids, jnp.full((n_pad - n, l), _PAD_IDX, dtype=jnp.int32)], axis=0)
    ids_l = ids.reshape(nb, tile_n, l).transpose(0, 2, 1).reshape(nb, 1, l * tile_n)

    embt = embedding.T
    w2t = jnp.transpose(conv_w, (2, 1, 0)).reshape(k * c, e).astype(jnp.bfloat16)
    bc = conv_b.reshape(1, e)
    wpt = proj_w.T.astype(jnp.bfloat16)
    wgt = gate_w.T.astype(jnp.bfloat16)
    bp = proj_b.reshape(1, e)
    bg = gate_b.reshape(1, e)

    kern = functools.partial(_fused_kernel, ksize=k, t_out=t_out, vocab=vcb,
                             tile_n=tile_n, c_dim=c)

    out = pl.pallas_call(
        kern,
        out_shape=jax.ShapeDtypeStruct((n_pad, e), jnp.float32),
        grid_spec=pltpu.PrefetchScalarGridSpec(
            num_scalar_prefetch=0,
            grid=(nb,),
            in_specs=[
                pl.BlockSpec((1, 1, l * tile_n), lambda i: (i, 0, 0)),
                pl.BlockSpec((c, vcb), lambda i: (0, 0)),
                pl.BlockSpec((k * c, e), lambda i: (0, 0)),
                pl.BlockSpec((1, e), lambda i: (0, 0)),
                pl.BlockSpec((e, e), lambda i: (0, 0)),
                pl.BlockSpec((1, e), lambda i: (0, 0)),
                pl.BlockSpec((e, e), lambda i: (0, 0)),
                pl.BlockSpec((1, e), lambda i: (0, 0)),
            ],
            out_specs=pl.BlockSpec((tile_n, e), lambda i: (i, 0)),
        ),
        compiler_params=pltpu.CompilerParams(
            dimension_semantics=("parallel",),
            vmem_limit_bytes=64 * 1024 * 1024,
        ),
    )(ids_l, embt, w2t, bc, wpt, bp, wgt, bg)

    return out[:n].reshape(s_len, b_size, e)


# speedup vs baseline: 4.0188x; 1.0593x over previous
import functools

import jax
import jax.numpy as jnp
from jax.experimental import pallas as pl
from jax.experimental.pallas import tpu as pltpu

_CHAR_EMBED = 50
_MAX_WORD_LEN = 21
_KSIZE = 5
_PAD_IDX = 0
_C_PAD = 50


def _fused_kernel(ids_ref, embt_ref, w2t_ref, bc_ref, wpt_ref, bp_ref,
                  wgt_ref, bg_ref, o_ref, *,
                  ksize, t_out, vocab, tile_n, c_pad):
    p = (t_out + ksize - 1) * tile_n

    ids = ids_ref[0]
    iota_v = jax.lax.broadcasted_iota(jnp.int32, (vocab, p), 0)
    onehot = (iota_v == ids).astype(jnp.bfloat16)

    emb = jnp.dot(embt_ref[...], onehot,
                  preferred_element_type=jnp.float32)
    embb = emb.astype(jnp.bfloat16)

    dn = (((0,), (0,)), ((), ()))
    pooled = None
    for t in range(t_out):
        xt = jnp.concatenate(
            [embb[:, (t + kk) * tile_n:(t + kk + 1) * tile_n]
             for kk in range(ksize)], axis=0)
        ct = jax.lax.dot_general(xt, w2t_ref[...], dn,
                                 preferred_element_type=jnp.float32)
        pooled = ct if pooled is None else jnp.maximum(pooled, ct)

    cb = jnp.maximum(pooled + bc_ref[...], 0.0)
    cbb = cb.astype(jnp.bfloat16)

    projt = jnp.maximum(
        jnp.dot(cbb, wpt_ref[...],
                preferred_element_type=jnp.float32) + bp_ref[...], 0.0)
    gatet = jax.nn.sigmoid(
        jnp.dot(cbb, wgt_ref[...],
                preferred_element_type=jnp.float32) + bg_ref[...])

    o_ref[...] = cb + gatet * (projt - cb)


def kernel(char_ids, embedding, conv_w, conv_b, proj_w, proj_b, gate_w,
           gate_b, *, tile_n=2048):
    s_len, b_size, l = char_ids.shape
    assert l == _MAX_WORD_LEN
    n = s_len * b_size
    k = _KSIZE
    t_out = l - k + 1
    e = conv_w.shape[0]
    vcb = embedding.shape[0]
    c = embedding.shape[1]
    c_pad = _C_PAD

    n_pad = ((n + tile_n - 1) // tile_n) * tile_n
    nb = n_pad // tile_n
    ids = char_ids.reshape(n, l).astype(jnp.int32)
    if n_pad != n:
        ids = jnp.concatenate(
            [ids, jnp.full((n_pad - n, l), _PAD_IDX, dtype=jnp.int32)], axis=0)
    ids_l = ids.reshape(nb, tile_n, l).transpose(0, 2, 1).reshape(nb, 1, l * tile_n)

    embt = jnp.zeros((c_pad, vcb), jnp.float32).at[:c].set(embedding.T).astype(jnp.bfloat16)
    w2t = jnp.zeros((k, c_pad, e), jnp.float32)
    w2t = w2t.at[:, :c, :].set(jnp.transpose(conv_w, (2, 1, 0)))
    w2t = w2t.reshape(k * c_pad, e).astype(jnp.bfloat16)
    bc = conv_b.reshape(1, e)
    wpt = proj_w.T.astype(jnp.bfloat16)
    wgt = gate_w.T.astype(jnp.bfloat16)
    bp = proj_b.reshape(1, e)
    bg = gate_b.reshape(1, e)

    kern = functools.partial(_fused_kernel, ksize=k, t_out=t_out, vocab=vcb,
                             tile_n=tile_n, c_pad=c_pad)

    out = pl.pallas_call(
        kern,
        out_shape=jax.ShapeDtypeStruct((n_pad, e), jnp.float32),
        grid_spec=pltpu.PrefetchScalarGridSpec(
            num_scalar_prefetch=0,
            grid=(nb,),
            in_specs=[
                pl.BlockSpec((1, 1, l * tile_n), lambda i: (i, 0, 0)),
                pl.BlockSpec((c_pad, vcb), lambda i: (0, 0)),
                pl.BlockSpec((k * c_pad, e), lambda i: (0, 0)),
                pl.BlockSpec((1, e), lambda i: (0, 0)),
                pl.BlockSpec((e, e), lambda i: (0, 0)),
                pl.BlockSpec((1, e), lambda i: (0, 0)),
                pl.BlockSpec((e, e), lambda i: (0, 0)),
                pl.BlockSpec((1, e), lambda i: (0, 0)),
            ],
            out_specs=pl.BlockSpec((tile_n, e), lambda i: (i, 0)),
        ),
        compiler_params=pltpu.CompilerParams(
            dimension_semantics=("parallel",),
            vmem_limit_bytes=64 * 1024 * 1024,
        ),
    )(ids_l, embt, w2t, bc, wpt, bp, wgt, bg)

    return out[:n].reshape(s_len, b_size, e)


# speedup vs baseline: 4.2391x; 1.0548x over previous
import functools

import jax
import jax.numpy as jnp
from jax.experimental import pallas as pl
from jax.experimental.pallas import tpu as pltpu

_CHAR_EMBED = 50
_MAX_WORD_LEN = 21
_KSIZE = 5
_PAD_IDX = 0
_C_PAD = 50


def _fused_kernel(ids_ref, embt_ref, w2t_ref, bc_ref, wpt_ref, bp_ref,
                  wgt_ref, bg_ref, o_ref, *,
                  ksize, t_out, vocab, tile_n, c_pad):
    p = (t_out + ksize - 1) * tile_n

    ids = ids_ref[0]
    iota_v = jax.lax.broadcasted_iota(jnp.int32, (vocab, p), 0)
    onehot = (iota_v == ids).astype(jnp.bfloat16)

    emb = jnp.dot(embt_ref[...], onehot,
                  preferred_element_type=jnp.float32)
    embb = emb.astype(jnp.bfloat16)

    dn = (((0,), (0,)), ((), ()))
    pooled = None
    for t in range(t_out):
        xt = jnp.concatenate(
            [embb[:, (t + kk) * tile_n:(t + kk + 1) * tile_n]
             for kk in range(ksize)], axis=0)
        ct = jax.lax.dot_general(xt, w2t_ref[...], dn,
                                 preferred_element_type=jnp.float32)
        pooled = ct if pooled is None else jnp.maximum(pooled, ct)

    cb = jnp.maximum(pooled + bc_ref[...], 0.0)
    cbb = cb.astype(jnp.bfloat16)

    projt = jnp.maximum(
        jnp.dot(cbb, wpt_ref[...],
                preferred_element_type=jnp.float32) + bp_ref[...], 0.0)
    gatet = jax.nn.sigmoid(
        jnp.dot(cbb, wgt_ref[...],
                preferred_element_type=jnp.float32) + bg_ref[...])

    o_ref[...] = cb + gatet * (projt - cb)


def kernel(char_ids, embedding, conv_w, conv_b, proj_w, proj_b, gate_w,
           gate_b, *, tile_n=4096):
    s_len, b_size, l = char_ids.shape
    assert l == _MAX_WORD_LEN
    n = s_len * b_size
    k = _KSIZE
    t_out = l - k + 1
    e = conv_w.shape[0]
    vcb = embedding.shape[0]
    c = embedding.shape[1]
    c_pad = _C_PAD

    n_pad = ((n + tile_n - 1) // tile_n) * tile_n
    nb = n_pad // tile_n
    ids = char_ids.reshape(n, l).astype(jnp.int32)
    if n_pad != n:
        ids = jnp.concatenate(
            [ids, jnp.full((n_pad - n, l), _PAD_IDX, dtype=jnp.int32)], axis=0)
    ids_l = ids.reshape(nb, tile_n, l).transpose(0, 2, 1).reshape(nb, 1, l * tile_n)

    embt = jnp.zeros((c_pad, vcb), jnp.float32).at[:c].set(embedding.T).astype(jnp.bfloat16)
    w2t = jnp.zeros((k, c_pad, e), jnp.float32)
    w2t = w2t.at[:, :c, :].set(jnp.transpose(conv_w, (2, 1, 0)))
    w2t = w2t.reshape(k * c_pad, e).astype(jnp.bfloat16)
    bc = conv_b.reshape(1, e)
    wpt = proj_w.T.astype(jnp.bfloat16)
    wgt = gate_w.T.astype(jnp.bfloat16)
    bp = proj_b.reshape(1, e)
    bg = gate_b.reshape(1, e)

    kern = functools.partial(_fused_kernel, ksize=k, t_out=t_out, vocab=vcb,
                             tile_n=tile_n, c_pad=c_pad)

    out = pl.pallas_call(
        kern,
        out_shape=jax.ShapeDtypeStruct((n_pad, e), jnp.float32),
        grid_spec=pltpu.PrefetchScalarGridSpec(
            num_scalar_prefetch=0,
            grid=(nb,),
            in_specs=[
                pl.BlockSpec((1, 1, l * tile_n), lambda i: (i, 0, 0)),
                pl.BlockSpec((c_pad, vcb), lambda i: (0, 0)),
                pl.BlockSpec((k * c_pad, e), lambda i: (0, 0)),
                pl.BlockSpec((1, e), lambda i: (0, 0)),
                pl.BlockSpec((e, e), lambda i: (0, 0)),
                pl.BlockSpec((1, e), lambda i: (0, 0)),
                pl.BlockSpec((e, e), lambda i: (0, 0)),
                pl.BlockSpec((1, e), lambda i: (0, 0)),
            ],
            out_specs=pl.BlockSpec((tile_n, e), lambda i: (i, 0)),
        ),
        compiler_params=pltpu.CompilerParams(
            dimension_semantics=("parallel",),
            vmem_limit_bytes=64 * 1024 * 1024,
        ),
    )(ids_l, embt, w2t, bc, wpt, bp, wgt, bg)

    return out[:n].reshape(s_len, b_size, e)
